# baseline (speedup 1.0000x reference)
import functools

import jax
import jax.numpy as jnp
from jax import lax
from jax.experimental import pallas as pl
from jax.experimental.pallas import tpu as pltpu
from jax.experimental.pallas import tpu_sc as plsc

N = 10000
E = 320000
G = 16
NP = 10240
DUMMY = N
NSUB = 16
NCORE = 2
CW = 128
ECW = 128
CHUNKS = 160
NBUF = 5
PER_SUB = CHUNKS * ECW
E_PAD = PER_SUB * NSUB
PER_W = E_PAD // 32
RB = 1280

_mesh = plsc.VectorSubcoreMesh(core_axis_name="c", subcore_axis_name="s")
_sc_params = pltpu.CompilerParams(needs_layout_passes=False,
                                  use_tc_tiling_on_sc=False)


@functools.partial(
    pl.kernel,
    mesh=_mesh,
    out_type=jax.ShapeDtypeStruct((32, NP), jnp.float32),
    compiler_params=_sc_params,
    scratch_types=[
        pltpu.VMEM((PER_W,), jnp.int32),
        pltpu.VMEM((NP,), jnp.float32),
    ],
)
def _deg_call(col_hbm, out_hbm, colv, hist):
    c = lax.axis_index("c")
    s = lax.axis_index("s")
    w = s * NCORE + c
    zeros16 = jnp.zeros((16,), jnp.float32)

    def zero_step(k, _):
        hist[pl.ds(k * 16, 16)] = zeros16
        return 0

    lax.fori_loop(0, NP // 16, zero_step, 0)
    pltpu.sync_copy(col_hbm.at[w], colv)
    ones16 = jnp.ones((16,), jnp.float32)

    def add_step(g, _):
        idx = colv[pl.ds(g * 16, 16)]
        plsc.addupdate_scatter(hist, [idx], ones16)
        return 0

    lax.fori_loop(0, PER_W // 16, add_step, 0)
    pltpu.sync_copy(hist, out_hbm.at[w])


QW = CW // 2


@functools.partial(
    pl.kernel,
    mesh=_mesh,
    out_type=jax.ShapeDtypeStruct((NCORE, NP, CW), jnp.float32),
    compiler_params=_sc_params,
    scratch_types=[
        pltpu.VMEM((CHUNKS, ECW), jnp.int32),
        pltpu.VMEM((CHUNKS, ECW), jnp.int32),
        [pltpu.VMEM((ECW, QW), jnp.float32)] * NBUF,
        pltpu.VMEM((64, QW), jnp.float32),
        pltpu.VMEM_SHARED((NP, QW), jnp.float32),
        [pltpu.SemaphoreType.DMA] * NBUF,
    ],
)
def _scat_call(y_hbm, row_hbm, col_hbm, out_hbm, rowv, colv, bufs, zbuf, acc,
               gsems):
    c = lax.axis_index("c")
    s = lax.axis_index("s")
    zeros16 = jnp.zeros((16,), jnp.float32)

    def zb_step(r, _):
        for j in range(QW // 16):
            zbuf[r, pl.ds(j * 16, 16)] = zeros16
        return 0

    lax.fori_loop(0, 64, zb_step, 0)
    pltpu.sync_copy(col_hbm.at[s], colv)

    for p in range(2):
        q = c * 2 + p

        def za_step(k, _):
            pltpu.sync_copy(zbuf, acc.at[pl.ds(s * 640 + k * 64, 64)])
            return 0

        lax.fori_loop(0, 10, za_step, 0)
        pltpu.sync_copy(row_hbm.at[q, s], rowv)
        plsc.subcore_barrier()

        for b in range(NBUF):
            pltpu.async_copy(y_hbm.at[rowv.at[b]], bufs[b], gsems[b])

        def step(g, _):
            for b in range(NBUF):
                j = g * NBUF + b
                pltpu.make_async_copy(y_hbm.at[rowv.at[j]], bufs[b],
                                      gsems[b]).wait()
                pltpu.sync_copy(bufs[b], acc.at[colv.at[j]], add=True)

                @pl.when(g < CHUNKS // NBUF - 1)
                def _():
                    pltpu.async_copy(y_hbm.at[rowv.at[j + NBUF]], bufs[b],
                                     gsems[b])
            return 0

        lax.fori_loop(0, CHUNKS // NBUF, step, 0)
        plsc.subcore_barrier()
        pltpu.sync_copy(
            acc.at[pl.ds(s * 640, 640)],
            out_hbm.at[c, pl.ds(s * 640, 640), pl.ds(p * QW, QW)])


_ONES32 = None


def _dis_from_parts(dp):
    ones = jnp.ones((32, 1), jnp.float32)
    deg = lax.dot_general(dp, ones, (((0,), (0,)), ((), ())),
                          preferred_element_type=jnp.float32) + 1.0
    return lax.rsqrt(deg)


def _b1_body(x_ref, w_ref, dp_ref, y_ref):
    dis = _dis_from_parts(dp_ref[...])
    xw = jnp.dot(x_ref[...], w_ref[...], preferred_element_type=jnp.float32)
    y = xw * dis
    y_ref[0] = y[:, :CW]
    y_ref[1] = y[:, CW:]


def _mid_body(acc_ref, yp_ref, dp_ref, b_ref, w_ref, y_ref):
    dis = _dis_from_parts(dp_ref[...])
    b = b_ref[...]
    hl = jnp.maximum(dis * (acc_ref[0] + yp_ref[0]) + b[:, :CW], 0.0)
    hr = jnp.maximum(dis * (acc_ref[1] + yp_ref[1]) + b[:, CW:], 0.0)
    h = jnp.concatenate([hl, hr], axis=1)
    xw = jnp.dot(h, w_ref[...], preferred_element_type=jnp.float32)
    y = xw * dis
    y_ref[0] = y[:, :CW]
    y_ref[1] = y[:, CW:]


def _final_body(acc_ref, yp_ref, dp_ref, b_ref, bat_ref, f1w_ref, f1b_ref,
                f2w_ref, f2b_ref, out_ref, pool, cnt):
    i = pl.program_id(0)
    dis = _dis_from_parts(dp_ref[...])
    b = b_ref[...]
    hl = jnp.maximum(dis * (acc_ref[0] + yp_ref[0]) + b[:, :CW], 0.0)
    hr = jnp.maximum(dis * (acc_ref[1] + yp_ref[1]) + b[:, CW:], 0.0)
    h = jnp.concatenate([hl, hr], axis=1)
    bat = bat_ref[...]
    iota = lax.broadcasted_iota(jnp.int32, (RB, G), 1)
    oh = (bat == iota).astype(jnp.float32)

    @pl.when(i == 0)
    def _():
        pool[...] = jnp.zeros_like(pool)
        cnt[...] = jnp.zeros_like(cnt)

    pool[...] += lax.dot_general(oh, h, (((0,), (0,)), ((), ())),
                                 preferred_element_type=jnp.float32)
    cnt[...] += lax.dot_general(oh, jnp.ones((RB, 1), jnp.float32),
                                (((0,), (0,)), ((), ())),
                                preferred_element_type=jnp.float32)

    @pl.when(i == NP // RB - 1)
    def _():
        hg = pool[...] / jnp.maximum(cnt[...], 1.0)
        fh = jnp.maximum(jnp.dot(hg, f1w_ref[...],
                                 preferred_element_type=jnp.float32)
                         + f1b_ref[...], 0.0)
        out_ref[...] = (jnp.dot(fh, f2w_ref[...],
                                preferred_element_type=jnp.float32)
                        + f2b_ref[...])


_GRID = (NP // RB,)
_f32 = jnp.float32


def _b1_pallas(x, w1, dp):
    return pl.pallas_call(
        _b1_body,
        grid=_GRID,
        in_specs=[
            pl.BlockSpec((RB, 128), lambda i: (i, 0)),
            pl.BlockSpec((128, 256), lambda i: (0, 0)),
            pl.BlockSpec((32, RB), lambda i: (0, i)),
        ],
        out_specs=pl.BlockSpec((NCORE, RB, CW), lambda i: (0, i, 0)),
        out_shape=jax.ShapeDtypeStruct((NCORE, NP, CW), _f32),
    )(x, w1, dp)


def _mid_pallas(acc, yp, dp, b, w):
    return pl.pallas_call(
        _mid_body,
        grid=_GRID,
        in_specs=[
            pl.BlockSpec((NCORE, RB, CW), lambda i: (0, i, 0)),
            pl.BlockSpec((NCORE, RB, CW), lambda i: (0, i, 0)),
            pl.BlockSpec((32, RB), lambda i: (0, i)),
            pl.BlockSpec((1, 256), lambda i: (0, 0)),
            pl.BlockSpec((256, 256), lambda i: (0, 0)),
        ],
        out_specs=pl.BlockSpec((NCORE, RB, CW), lambda i: (0, i, 0)),
        out_shape=jax.ShapeDtypeStruct((NCORE, NP, CW), _f32),
    )(acc, yp, dp, b, w)


def _final_pallas(acc, yp, dp, b, bat, f1w, f1b, f2w, f2b):
    return pl.pallas_call(
        _final_body,
        grid=_GRID,
        in_specs=[
            pl.BlockSpec((NCORE, RB, CW), lambda i: (0, i, 0)),
            pl.BlockSpec((NCORE, RB, CW), lambda i: (0, i, 0)),
            pl.BlockSpec((32, RB), lambda i: (0, i)),
            pl.BlockSpec((1, 256), lambda i: (0, 0)),
            pl.BlockSpec((RB, 1), lambda i: (i, 0)),
            pl.BlockSpec((256, 128), lambda i: (0, 0)),
            pl.BlockSpec((1, 128), lambda i: (0, 0)),
            pl.BlockSpec((128, 10), lambda i: (0, 0)),
            pl.BlockSpec((1, 10), lambda i: (0, 0)),
        ],
        out_specs=pl.BlockSpec((G, 10), lambda i: (0, 0)),
        out_shape=jax.ShapeDtypeStruct((G, 10), _f32),
        scratch_shapes=[
            pltpu.VMEM((G, 256), _f32),
            pltpu.VMEM((G, 1), _f32),
        ],
    )(acc, yp, dp, b, bat, f1w, f1b, f2w, f2b)


def kernel(x, edge_index, batch, W1, b1, W2, b2, W3, b3,
           fc1_W, fc1_b, fc2_W, fc2_b):
    row = edge_index[0].astype(jnp.int32)
    col = edge_index[1].astype(jnp.int32)
    pad = E_PAD - E
    pad_ar = jnp.arange(pad, dtype=jnp.int32)
    row_p = jnp.concatenate([row, pad_ar % N])
    col_p = jnp.concatenate([col, DUMMY + pad_ar % (NP - N)])
    row3 = row_p.reshape(NSUB, CHUNKS, ECW)
    col3 = col_p.reshape(NSUB, CHUNKS, ECW)
    off = jnp.array([0, 1, 2 * NP, 2 * NP + 1], jnp.int32)[:, None, None, None]
    row5 = (2 * row3)[None] + off
    colA = col_p.reshape(32, PER_W)

    x_pad = jnp.concatenate([x, jnp.zeros((NP - N, x.shape[1]), _f32)])
    bat_p = jnp.concatenate([batch.astype(jnp.int32),
                             jnp.full((NP - N,), G, jnp.int32)]).reshape(NP, 1)
    b1r = b1.reshape(1, 256)
    b2r = b2.reshape(1, 256)
    b3r = b3.reshape(1, 256)
    f1br = fc1_b.reshape(1, 128)
    f2br = fc2_b.reshape(1, 10)

    dp = _deg_call(colA)
    y1 = _b1_pallas(x_pad, W1, dp)
    a1 = _scat_call(y1.reshape(NCORE * NP * 2, QW), row5, col3)
    y2 = _mid_pallas(a1, y1, dp, b1r, W2)
    a2 = _scat_call(y2.reshape(NCORE * NP * 2, QW), row5, col3)
    y3 = _mid_pallas(a2, y2, dp, b2r, W3)
    a3 = _scat_call(y3.reshape(NCORE * NP * 2, QW), row5, col3)
    return _final_pallas(a3, y3, dp, b3r, bat_p, fc1_W, f1br, fc2_W, f2br)


# speedup vs baseline: 1.2728x; 1.2728x over previous
import functools

import jax
import jax.numpy as jnp
from jax import lax
from jax.experimental import pallas as pl
from jax.experimental.pallas import tpu as pltpu
from jax.experimental.pallas import tpu_sc as plsc

N = 10000
E = 320000
G = 16
NP = 10240
DUMMY = N
NSUB = 16
NCORE = 2
CW = 128
ECW = 128
CHUNKS = 160
NBUF = 4
PER_SUB = CHUNKS * ECW
E_PAD = PER_SUB * NSUB
PER_W = E_PAD // 32
RB = 1280

_mesh = plsc.VectorSubcoreMesh(core_axis_name="c", subcore_axis_name="s")
_sc_params = pltpu.CompilerParams(needs_layout_passes=False,
                                  use_tc_tiling_on_sc=False)
_f32 = jnp.float32
_bf16 = jnp.bfloat16


@functools.partial(
    pl.kernel,
    mesh=_mesh,
    out_type=jax.ShapeDtypeStruct((32, NP), jnp.float32),
    compiler_params=_sc_params,
    scratch_types=[
        pltpu.VMEM((PER_W,), jnp.int32),
        pltpu.VMEM((NP,), jnp.float32),
    ],
)
def _deg_call(col_hbm, out_hbm, colv, hist):
    c = lax.axis_index("c")
    s = lax.axis_index("s")
    w = s * NCORE + c
    zeros16 = jnp.zeros((16,), jnp.float32)

    def zero_step(k, _):
        hist[pl.ds(k * 16, 16)] = zeros16
        return 0

    lax.fori_loop(0, NP // 16, zero_step, 0)
    pltpu.sync_copy(col_hbm.at[w], colv)
    ones16 = jnp.ones((16,), jnp.float32)

    def add_step(g, _):
        idx = colv[pl.ds(g * 16, 16)]
        plsc.addupdate_scatter(hist, [idx], ones16)
        return 0

    lax.fori_loop(0, PER_W // 16, add_step, 0)
    pltpu.sync_copy(hist, out_hbm.at[w])


@functools.partial(
    pl.kernel,
    mesh=_mesh,
    out_type=jax.ShapeDtypeStruct((NCORE, NP, CW), jnp.bfloat16),
    compiler_params=_sc_params,
    scratch_types=[
        pltpu.VMEM((CHUNKS, ECW), jnp.int32),
        pltpu.VMEM((CHUNKS, ECW), jnp.int32),
        [pltpu.VMEM((ECW, CW), jnp.bfloat16)] * NBUF,
        pltpu.VMEM_SHARED((NP, CW), jnp.bfloat16),
        [pltpu.SemaphoreType.DMA] * NBUF,
    ],
)
def _scat_call(y_hbm, row_hbm, col_hbm, z_hbm, out_hbm, rowv, colv, bufs, acc,
               sems):
    c = lax.axis_index("c")
    s = lax.axis_index("s")
    pltpu.sync_copy(col_hbm.at[s], colv)
    pltpu.sync_copy(row_hbm.at[c, s], rowv)
    pltpu.sync_copy(z_hbm, acc.at[pl.ds(s * 640, 640)])
    plsc.subcore_barrier()

    for b in range(NBUF):
        pltpu.async_copy(y_hbm.at[rowv.at[b]], bufs[b], sems[b])

    def step(g, _):
        for b in range(NBUF):
            j = g * NBUF + b
            pltpu.make_async_copy(y_hbm.at[rowv.at[j]], bufs[b],
                                  sems[b]).wait()
            pltpu.sync_copy(bufs[b], acc.at[colv.at[j]], add=True)

            @pl.when(g < CHUNKS // NBUF - 1)
            def _():
                pltpu.async_copy(y_hbm.at[rowv.at[j + NBUF]], bufs[b],
                                 sems[b])
        return 0

    lax.fori_loop(0, CHUNKS // NBUF, step, 0)
    plsc.subcore_barrier()
    pltpu.sync_copy(acc.at[pl.ds(s * 640, 640)],
                    out_hbm.at[c, pl.ds(s * 640, 640)])


def _dis_from_parts(dp):
    ones = jnp.ones((32, 1), jnp.float32)
    deg = lax.dot_general(dp, ones, (((0,), (0,)), ((), ())),
                          preferred_element_type=jnp.float32) + 1.0
    return lax.rsqrt(deg)


def _b1_body(x_ref, w_ref, dp_ref, y_ref):
    dis = _dis_from_parts(dp_ref[...])
    xw = jnp.dot(x_ref[...], w_ref[...], preferred_element_type=jnp.float32)
    y = xw * dis
    y_ref[0] = y[:, :CW].astype(_bf16)
    y_ref[1] = y[:, CW:].astype(_bf16)


def _mid_body(acc_ref, yp_ref, dp_ref, b_ref, w_ref, y_ref):
    dis = _dis_from_parts(dp_ref[...])
    b = b_ref[...]
    hl = jnp.maximum(
        dis * (acc_ref[0].astype(_f32) + yp_ref[0].astype(_f32))
        + b[:, :CW], 0.0)
    hr = jnp.maximum(
        dis * (acc_ref[1].astype(_f32) + yp_ref[1].astype(_f32))
        + b[:, CW:], 0.0)
    h = jnp.concatenate([hl, hr], axis=1)
    xw = jnp.dot(h, w_ref[...], preferred_element_type=jnp.float32)
    y = xw * dis
    y_ref[0] = y[:, :CW].astype(_bf16)
    y_ref[1] = y[:, CW:].astype(_bf16)


def _final_body(acc_ref, yp_ref, dp_ref, b_ref, bat_ref, f1w_ref, f1b_ref,
                f2w_ref, f2b_ref, out_ref, pool, cnt):
    i = pl.program_id(0)
    dis = _dis_from_parts(dp_ref[...])
    b = b_ref[...]
    hl = jnp.maximum(
        dis * (acc_ref[0].astype(_f32) + yp_ref[0].astype(_f32))
        + b[:, :CW], 0.0)
    hr = jnp.maximum(
        dis * (acc_ref[1].astype(_f32) + yp_ref[1].astype(_f32))
        + b[:, CW:], 0.0)
    h = jnp.concatenate([hl, hr], axis=1)
    bat = bat_ref[...]
    iota = lax.broadcasted_iota(jnp.int32, (RB, G), 1)
    oh = (bat == iota).astype(jnp.float32)

    @pl.when(i == 0)
    def _():
        pool[...] = jnp.zeros_like(pool)
        cnt[...] = jnp.zeros_like(cnt)

    pool[...] += lax.dot_general(oh, h, (((0,), (0,)), ((), ())),
                                 preferred_element_type=jnp.float32)
    cnt[...] += lax.dot_general(oh, jnp.ones((RB, 1), jnp.float32),
                                (((0,), (0,)), ((), ())),
                                preferred_element_type=jnp.float32)

    @pl.when(i == NP // RB - 1)
    def _():
        hg = pool[...] / jnp.maximum(cnt[...], 1.0)
        fh = jnp.maximum(jnp.dot(hg, f1w_ref[...],
                                 preferred_element_type=jnp.float32)
                         + f1b_ref[...], 0.0)
        out_ref[...] = (jnp.dot(fh, f2w_ref[...],
                                preferred_element_type=jnp.float32)
                        + f2b_ref[...])


_GRID = (NP // RB,)


def _b1_pallas(x, w1, dp):
    return pl.pallas_call(
        _b1_body,
        grid=_GRID,
        in_specs=[
            pl.BlockSpec((RB, 128), lambda i: (i, 0)),
            pl.BlockSpec((128, 256), lambda i: (0, 0)),
            pl.BlockSpec((32, RB), lambda i: (0, i)),
        ],
        out_specs=pl.BlockSpec((NCORE, RB, CW), lambda i: (0, i, 0)),
        out_shape=jax.ShapeDtypeStruct((NCORE, NP, CW), _bf16),
    )(x, w1, dp)


def _mid_pallas(acc, yp, dp, b, w):
    return pl.pallas_call(
        _mid_body,
        grid=_GRID,
        in_specs=[
            pl.BlockSpec((NCORE, RB, CW), lambda i: (0, i, 0)),
            pl.BlockSpec((NCORE, RB, CW), lambda i: (0, i, 0)),
            pl.BlockSpec((32, RB), lambda i: (0, i)),
            pl.BlockSpec((1, 256), lambda i: (0, 0)),
            pl.BlockSpec((256, 256), lambda i: (0, 0)),
        ],
        out_specs=pl.BlockSpec((NCORE, RB, CW), lambda i: (0, i, 0)),
        out_shape=jax.ShapeDtypeStruct((NCORE, NP, CW), _bf16),
    )(acc, yp, dp, b, w)


def _final_pallas(acc, yp, dp, b, bat, f1w, f1b, f2w, f2b):
    return pl.pallas_call(
        _final_body,
        grid=_GRID,
        in_specs=[
            pl.BlockSpec((NCORE, RB, CW), lambda i: (0, i, 0)),
            pl.BlockSpec((NCORE, RB, CW), lambda i: (0, i, 0)),
            pl.BlockSpec((32, RB), lambda i: (0, i)),
            pl.BlockSpec((1, 256), lambda i: (0, 0)),
            pl.BlockSpec((RB, 1), lambda i: (i, 0)),
            pl.BlockSpec((256, 128), lambda i: (0, 0)),
            pl.BlockSpec((1, 128), lambda i: (0, 0)),
            pl.BlockSpec((128, 10), lambda i: (0, 0)),
            pl.BlockSpec((1, 10), lambda i: (0, 0)),
        ],
        out_specs=pl.BlockSpec((G, 10), lambda i: (0, 0)),
        out_shape=jax.ShapeDtypeStruct((G, 10), _f32),
        scratch_shapes=[
            pltpu.VMEM((G, 256), _f32),
            pltpu.VMEM((G, 1), _f32),
        ],
    )(acc, yp, dp, b, bat, f1w, f1b, f2w, f2b)


def kernel(x, edge_index, batch, W1, b1, W2, b2, W3, b3,
           fc1_W, fc1_b, fc2_W, fc2_b):
    row = edge_index[0].astype(jnp.int32)
    col = edge_index[1].astype(jnp.int32)
    pad = E_PAD - E
    pad_ar = jnp.arange(pad, dtype=jnp.int32)
    row_p = jnp.concatenate([row, pad_ar % N])
    col_p = jnp.concatenate([col, DUMMY + pad_ar % (NP - N)])
    row3 = row_p.reshape(NSUB, CHUNKS, ECW)
    col3 = col_p.reshape(NSUB, CHUNKS, ECW)
    off = jnp.array([0, NP], jnp.int32)[:, None, None, None]
    row4 = row3[None] + off
    colA = col_p.reshape(32, PER_W)

    x_pad = jnp.concatenate([x, jnp.zeros((NP - N, x.shape[1]), _f32)])
    bat_p = jnp.concatenate([batch.astype(jnp.int32),
                             jnp.full((NP - N,), G, jnp.int32)]).reshape(NP, 1)
    zrows = jnp.zeros((640, CW), _bf16)
    b1r = b1.reshape(1, 256)
    b2r = b2.reshape(1, 256)
    b3r = b3.reshape(1, 256)
    f1br = fc1_b.reshape(1, 128)
    f2br = fc2_b.reshape(1, 10)

    dp = _deg_call(colA)
    y1 = _b1_pallas(x_pad, W1, dp)
    a1 = _scat_call(y1.reshape(NCORE * NP, CW), row4, col3, zrows)
    y2 = _mid_pallas(a1, y1, dp, b1r, W2)
    a2 = _scat_call(y2.reshape(NCORE * NP, CW), row4, col3, zrows)
    y3 = _mid_pallas(a2, y2, dp, b2r, W3)
    a3 = _scat_call(y3.reshape(NCORE * NP, CW), row4, col3, zrows)
    return _final_pallas(a3, y3, dp, b3r, bat_p, fc1_W, f1br, fc2_W, f2br)


# speedup vs baseline: 1.2967x; 1.0188x over previous
import functools

import jax
import jax.numpy as jnp
from jax import lax
from jax.experimental import pallas as pl
from jax.experimental.pallas import tpu as pltpu
from jax.experimental.pallas import tpu_sc as plsc

N = 10000
E = 320000
G = 16
NP = 10240
DUMMY = N
NSUB = 16
NCORE = 2
CW = 128
ECW = 128
CHUNKS = 160
NBUF = 4
PER_SUB = CHUNKS * ECW
E_PAD = PER_SUB * NSUB
PER_W = E_PAD // 32
RB = 2560

_mesh = plsc.VectorSubcoreMesh(core_axis_name="c", subcore_axis_name="s")
_sc_params = pltpu.CompilerParams(needs_layout_passes=False,
                                  use_tc_tiling_on_sc=False)
_f32 = jnp.float32
_bf16 = jnp.bfloat16


@functools.partial(
    pl.kernel,
    mesh=_mesh,
    out_type=jax.ShapeDtypeStruct((32, NP), jnp.float32),
    compiler_params=_sc_params,
    scratch_types=[
        pltpu.VMEM((PER_W,), jnp.int32),
        pltpu.VMEM((NP,), jnp.float32),
    ],
)
def _deg_call(col_hbm, out_hbm, colv, hist):
    c = lax.axis_index("c")
    s = lax.axis_index("s")
    w = s * NCORE + c
    zeros16 = jnp.zeros((16,), jnp.float32)

    def zero_step(k, _):
        hist[pl.ds(k * 16, 16)] = zeros16
        return 0

    lax.fori_loop(0, NP // 16, zero_step, 0)
    pltpu.sync_copy(col_hbm.at[w], colv)
    ones16 = jnp.ones((16,), jnp.float32)

    def add_step(g, _):
        idx = colv[pl.ds(g * 16, 16)]
        plsc.addupdate_scatter(hist, [idx], ones16)
        return 0

    lax.fori_loop(0, PER_W // 16, add_step, 0)
    pltpu.sync_copy(hist, out_hbm.at[w])


@functools.partial(
    pl.kernel,
    mesh=_mesh,
    out_type=jax.ShapeDtypeStruct((NCORE, NP, CW), jnp.bfloat16),
    compiler_params=_sc_params,
    scratch_types=[
        pltpu.VMEM((CHUNKS, ECW), jnp.int32),
        pltpu.VMEM((CHUNKS, ECW), jnp.int32),
        [pltpu.VMEM((ECW, CW), jnp.bfloat16)] * NBUF,
        pltpu.VMEM_SHARED((NP, CW), jnp.bfloat16),
        [pltpu.SemaphoreType.DMA] * NBUF,
    ],
)
def _scat_call(y_hbm, row_hbm, col_hbm, z_hbm, out_hbm, rowv, colv, bufs, acc,
               sems):
    c = lax.axis_index("c")
    s = lax.axis_index("s")
    pltpu.sync_copy(col_hbm.at[s], colv)
    pltpu.sync_copy(row_hbm.at[c, s], rowv)
    pltpu.sync_copy(z_hbm, acc.at[pl.ds(s * 640, 640)])
    plsc.subcore_barrier()

    for b in range(NBUF):
        pltpu.async_copy(y_hbm.at[rowv.at[b]], bufs[b], sems[b])

    def step(g, _):
        for b in range(NBUF):
            j = g * NBUF + b
            pltpu.make_async_copy(y_hbm.at[rowv.at[j]], bufs[b],
                                  sems[b]).wait()
            pltpu.sync_copy(bufs[b], acc.at[colv.at[j]], add=True)

            @pl.when(g < CHUNKS // NBUF - 1)
            def _():
                pltpu.async_copy(y_hbm.at[rowv.at[j + NBUF]], bufs[b],
                                 sems[b])
        return 0

    lax.fori_loop(0, CHUNKS // NBUF, step, 0)
    plsc.subcore_barrier()
    pltpu.sync_copy(acc.at[pl.ds(s * 640, 640)],
                    out_hbm.at[c, pl.ds(s * 640, 640)])


def _dis_from_parts(dp):
    ones = jnp.ones((32, 1), jnp.float32)
    deg = lax.dot_general(dp, ones, (((0,), (0,)), ((), ())),
                          preferred_element_type=jnp.float32) + 1.0
    return lax.rsqrt(deg)


def _b1_body(x_ref, w_ref, dp_ref, y_ref):
    dis = _dis_from_parts(dp_ref[...])
    xw = jnp.dot(x_ref[...], w_ref[...], preferred_element_type=jnp.float32)
    y = xw * dis
    y_ref[0] = y[:, :CW].astype(_bf16)
    y_ref[1] = y[:, CW:].astype(_bf16)


def _mid_body(acc_ref, yp_ref, dp_ref, b_ref, w_ref, y_ref):
    dis = _dis_from_parts(dp_ref[...])
    b = b_ref[...]
    hl = jnp.maximum(
        dis * (acc_ref[0].astype(_f32) + yp_ref[0].astype(_f32))
        + b[:, :CW], 0.0)
    hr = jnp.maximum(
        dis * (acc_ref[1].astype(_f32) + yp_ref[1].astype(_f32))
        + b[:, CW:], 0.0)
    h = jnp.concatenate([hl, hr], axis=1)
    xw = jnp.dot(h, w_ref[...], preferred_element_type=jnp.float32)
    y = xw * dis
    y_ref[0] = y[:, :CW].astype(_bf16)
    y_ref[1] = y[:, CW:].astype(_bf16)


def _final_body(acc_ref, yp_ref, dp_ref, b_ref, bat_ref, f1w_ref, f1b_ref,
                f2w_ref, f2b_ref, out_ref, pool, cnt):
    i = pl.program_id(0)
    dis = _dis_from_parts(dp_ref[...])
    b = b_ref[...]
    hl = jnp.maximum(
        dis * (acc_ref[0].astype(_f32) + yp_ref[0].astype(_f32))
        + b[:, :CW], 0.0)
    hr = jnp.maximum(
        dis * (acc_ref[1].astype(_f32) + yp_ref[1].astype(_f32))
        + b[:, CW:], 0.0)
    h = jnp.concatenate([hl, hr], axis=1)
    bat = bat_ref[...]
    iota = lax.broadcasted_iota(jnp.int32, (RB, G), 1)
    oh = (bat == iota).astype(jnp.float32)

    @pl.when(i == 0)
    def _():
        pool[...] = jnp.zeros_like(pool)
        cnt[...] = jnp.zeros_like(cnt)

    pool[...] += lax.dot_general(oh, h, (((0,), (0,)), ((), ())),
                                 preferred_element_type=jnp.float32)
    cnt[...] += lax.dot_general(oh, jnp.ones((RB, 1), jnp.float32),
                                (((0,), (0,)), ((), ())),
                                preferred_element_type=jnp.float32)

    @pl.when(i == NP // RB - 1)
    def _():
        hg = pool[...] / jnp.maximum(cnt[...], 1.0)
        fh = jnp.maximum(jnp.dot(hg, f1w_ref[...],
                                 preferred_element_type=jnp.float32)
                         + f1b_ref[...], 0.0)
        out_ref[...] = (jnp.dot(fh, f2w_ref[...],
                                preferred_element_type=jnp.float32)
                        + f2b_ref[...])


_GRID = (NP // RB,)


def _b1_pallas(x, w1, dp):
    return pl.pallas_call(
        _b1_body,
        grid=_GRID,
        in_specs=[
            pl.BlockSpec((RB, 128), lambda i: (i, 0)),
            pl.BlockSpec((128, 256), lambda i: (0, 0)),
            pl.BlockSpec((32, RB), lambda i: (0, i)),
        ],
        out_specs=pl.BlockSpec((NCORE, RB, CW), lambda i: (0, i, 0)),
        out_shape=jax.ShapeDtypeStruct((NCORE, NP, CW), _bf16),
    )(x, w1, dp)


def _mid_pallas(acc, yp, dp, b, w):
    return pl.pallas_call(
        _mid_body,
        grid=_GRID,
        in_specs=[
            pl.BlockSpec((NCORE, RB, CW), lambda i: (0, i, 0)),
            pl.BlockSpec((NCORE, RB, CW), lambda i: (0, i, 0)),
            pl.BlockSpec((32, RB), lambda i: (0, i)),
            pl.BlockSpec((1, 256), lambda i: (0, 0)),
            pl.BlockSpec((256, 256), lambda i: (0, 0)),
        ],
        out_specs=pl.BlockSpec((NCORE, RB, CW), lambda i: (0, i, 0)),
        out_shape=jax.ShapeDtypeStruct((NCORE, NP, CW), _bf16),
    )(acc, yp, dp, b, w)


def _final_pallas(acc, yp, dp, b, bat, f1w, f1b, f2w, f2b):
    return pl.pallas_call(
        _final_body,
        grid=_GRID,
        in_specs=[
            pl.BlockSpec((NCORE, RB, CW), lambda i: (0, i, 0)),
            pl.BlockSpec((NCORE, RB, CW), lambda i: (0, i, 0)),
            pl.BlockSpec((32, RB), lambda i: (0, i)),
            pl.BlockSpec((1, 256), lambda i: (0, 0)),
            pl.BlockSpec((RB, 1), lambda i: (i, 0)),
            pl.BlockSpec((256, 128), lambda i: (0, 0)),
            pl.BlockSpec((1, 128), lambda i: (0, 0)),
            pl.BlockSpec((128, 10), lambda i: (0, 0)),
            pl.BlockSpec((1, 10), lambda i: (0, 0)),
        ],
        out_specs=pl.BlockSpec((G, 10), lambda i: (0, 0)),
        out_shape=jax.ShapeDtypeStruct((G, 10), _f32),
        scratch_shapes=[
            pltpu.VMEM((G, 256), _f32),
            pltpu.VMEM((G, 1), _f32),
        ],
    )(acc, yp, dp, b, bat, f1w, f1b, f2w, f2b)


def kernel(x, edge_index, batch, W1, b1, W2, b2, W3, b3,
           fc1_W, fc1_b, fc2_W, fc2_b):
    row = edge_index[0].astype(jnp.int32)
    col = edge_index[1].astype(jnp.int32)
    pad = E_PAD - E
    pad_ar = jnp.arange(pad, dtype=jnp.int32)
    row_p = jnp.concatenate([row, pad_ar % N])
    col_p = jnp.concatenate([col, DUMMY + pad_ar % (NP - N)])
    row3 = row_p.reshape(NSUB, CHUNKS, ECW)
    col3 = col_p.reshape(NSUB, CHUNKS, ECW)
    off = jnp.array([0, NP], jnp.int32)[:, None, None, None]
    row4 = row3[None] + off
    colA = col_p.reshape(32, PER_W)

    x_pad = jnp.concatenate([x, jnp.zeros((NP - N, x.shape[1]), _f32)])
    bat_p = jnp.concatenate([batch.astype(jnp.int32),
                             jnp.full((NP - N,), G, jnp.int32)]).reshape(NP, 1)
    zrows = jnp.zeros((640, CW), _bf16)
    b1r = b1.reshape(1, 256)
    b2r = b2.reshape(1, 256)
    b3r = b3.reshape(1, 256)
    f1br = fc1_b.reshape(1, 128)
    f2br = fc2_b.reshape(1, 10)

    dp = _deg_call(colA)
    y1 = _b1_pallas(x_pad, W1, dp)
    a1 = _scat_call(y1.reshape(NCORE * NP, CW), row4, col3, zrows)
    y2 = _mid_pallas(a1, y1, dp, b1r, W2)
    a2 = _scat_call(y2.reshape(NCORE * NP, CW), row4, col3, zrows)
    y3 = _mid_pallas(a2, y2, dp, b2r, W3)
    a3 = _scat_call(y3.reshape(NCORE * NP, CW), row4, col3, zrows)
    return _final_pallas(a3, y3, dp, b3r, bat_p, fc1_W, f1br, fc2_W, f2br)
